# baseline (speedup 1.0000x reference)
import functools

import jax
import jax.numpy as jnp
from jax import lax
from jax.experimental import pallas as pl
from jax.experimental.pallas import tpu as pltpu
from jax.experimental.pallas import tpu_sc as plsc

B = 1024
D = 32
C = 100000
SCALAR_GAIN = 30.0
TILE_C = 2048
NUM_BLOCKS = (C + TILE_C - 1) // TILE_C


@functools.cache
def _make_sc_gather():
    info = plsc.get_sparse_core_info()
    nw = info.num_cores * info.num_subcores
    bpw = B // nw
    mesh = plsc.VectorSubcoreMesh(core_axis_name="c", subcore_axis_name="s")

    @functools.partial(
        pl.kernel,
        mesh=mesh,
        out_type=jax.ShapeDtypeStruct((B, D), jnp.float32),
        scratch_types=[
            pltpu.VMEM((bpw,), jnp.int32),
            pltpu.VMEM((bpw, D), jnp.float32),
            pltpu.SemaphoreType.DMA,
        ],
        compiler_params=pltpu.CompilerParams(use_tc_tiling_on_sc=False),
    )
    def gather_rows(lut_hbm, tgt_hbm, out_hbm, idx_v, rows_v, sem):
        wid = lax.axis_index("s") * info.num_cores + lax.axis_index("c")
        base = wid * bpw
        pltpu.sync_copy(tgt_hbm.at[pl.ds(base, bpw)], idx_v)
        pltpu.async_copy(lut_hbm.at[idx_v], rows_v, sem).wait()
        pltpu.sync_copy(rows_v, out_hbm.at[pl.ds(base, bpw)])

    return gather_rows


def _tc_body(x_ref, lut_ref, tgt_ref, out_ref, loss_ref, m_ref, s_ref):
    i = pl.program_id(0)
    x = x_ref[...]
    lt = lut_ref[...]
    logits = lax.dot_general(
        x, lt, (((1,), (1,)), ((), ())), preferred_element_type=jnp.float32
    ) * SCALAR_GAIN
    out_ref[...] = logits

    col = i * TILE_C + lax.broadcasted_iota(jnp.int32, (1, TILE_C), 1)
    masked = jnp.where(col < C, logits, -jnp.inf)
    tile_max = jnp.max(masked, axis=1, keepdims=True)

    @pl.when(i == 0)
    def _init():
        m_ref[...] = tile_max
        s_ref[...] = jnp.sum(jnp.exp(masked - tile_max), axis=1, keepdims=True)

    @pl.when(i > 0)
    def _update():
        m_old = m_ref[...]
        m_new = jnp.maximum(m_old, tile_max)
        s_ref[...] = s_ref[...] * jnp.exp(m_old - m_new) + jnp.sum(
            jnp.exp(masked - m_new), axis=1, keepdims=True
        )
        m_ref[...] = m_new

    @pl.when(i == NUM_BLOCKS - 1)
    def _finish():
        lse = m_ref[...] + jnp.log(s_ref[...])
        tgt_logit = (
            jnp.sum(x * tgt_ref[...], axis=1, keepdims=True) * SCALAR_GAIN
        )
        loss_ref[...] = jnp.mean(lse - tgt_logit, keepdims=True)


def kernel(inputs, targets, lut):
    tgt_rows = _make_sc_gather()(lut, targets)

    logits, loss = pl.pallas_call(
        _tc_body,
        grid=(NUM_BLOCKS,),
        in_specs=[
            pl.BlockSpec((B, D), lambda i: (0, 0)),
            pl.BlockSpec((TILE_C, D), lambda i: (i, 0)),
            pl.BlockSpec((B, D), lambda i: (0, 0)),
        ],
        out_specs=[
            pl.BlockSpec((B, TILE_C), lambda i: (0, i)),
            pl.BlockSpec((1, 1), lambda i: (0, 0)),
        ],
        out_shape=[
            jax.ShapeDtypeStruct((B, C), jnp.float32),
            jax.ShapeDtypeStruct((1, 1), jnp.float32),
        ],
        scratch_shapes=[
            pltpu.VMEM((B, 1), jnp.float32),
            pltpu.VMEM((B, 1), jnp.float32),
        ],
        compiler_params=pltpu.CompilerParams(
            dimension_semantics=("arbitrary",)
        ),
    )(inputs, lut, tgt_rows)

    return (loss[0, 0], logits)


# speedup vs baseline: 2.5171x; 2.5171x over previous
import functools

import jax
import jax.numpy as jnp
from jax import lax
from jax.experimental import pallas as pl
from jax.experimental.pallas import tpu as pltpu
from jax.experimental.pallas import tpu_sc as plsc

B = 1024
D = 32
C = 100000
SCALAR_GAIN = 30.0
TILE_R = 2048
NUM_BLOCKS = (C + TILE_R - 1) // TILE_R


@functools.cache
def _make_sc_gather():
    info = plsc.get_sparse_core_info()
    nw = info.num_cores * info.num_subcores
    bpw = B // nw
    mesh = plsc.VectorSubcoreMesh(core_axis_name="c", subcore_axis_name="s")

    @functools.partial(
        pl.kernel,
        mesh=mesh,
        out_type=jax.ShapeDtypeStruct((B, D), jnp.float32),
        scratch_types=[
            pltpu.VMEM((bpw,), jnp.int32),
            pltpu.VMEM((bpw, D), jnp.float32),
            pltpu.SemaphoreType.DMA,
        ],
        compiler_params=pltpu.CompilerParams(use_tc_tiling_on_sc=False),
    )
    def gather_rows(lut_hbm, tgt_hbm, out_hbm, idx_v, rows_v, sem):
        wid = lax.axis_index("s") * info.num_cores + lax.axis_index("c")
        base = wid * bpw
        pltpu.sync_copy(tgt_hbm.at[pl.ds(base, bpw)], idx_v)
        pltpu.async_copy(lut_hbm.at[idx_v], rows_v, sem).wait()
        pltpu.sync_copy(rows_v, out_hbm.at[pl.ds(base, bpw)])

    return gather_rows


def _tc_body(xT_ref, lutT_ref, tgt_ref, out_ref, loss_ref, m_ref, s_ref):
    i = pl.program_id(0)
    xT = xT_ref[...]
    lt = lutT_ref[...]
    logitsT = lax.dot_general(
        lt, xT, (((0,), (0,)), ((), ())), preferred_element_type=jnp.float32
    ) * SCALAR_GAIN
    out_ref[...] = logitsT

    row = i * TILE_R + lax.broadcasted_iota(jnp.int32, (TILE_R, 1), 0)
    masked = jnp.where(row < C, logitsT, -jnp.inf)
    tile_max = jnp.max(masked, axis=0, keepdims=True)

    @pl.when(i == 0)
    def _init():
        m_ref[...] = jnp.full((1, B), -jnp.inf, jnp.float32)
        s_ref[...] = jnp.zeros((1, B), jnp.float32)

    m_old = m_ref[...]
    m_new = jnp.maximum(m_old, tile_max)
    s_ref[...] = s_ref[...] * jnp.exp(m_old - m_new) + jnp.sum(
        jnp.exp(masked - m_new), axis=0, keepdims=True
    )
    m_ref[...] = m_new

    @pl.when(i == NUM_BLOCKS - 1)
    def _finish():
        lse = m_ref[...] + jnp.log(s_ref[...])
        tgt_logit = jnp.sum(xT * tgt_ref[...], axis=0, keepdims=True) * (
            SCALAR_GAIN
        )
        loss_ref[...] = jnp.mean(lse - tgt_logit, keepdims=True)


def kernel(inputs, targets, lut):
    tgt_rows = _make_sc_gather()(lut, targets)

    logitsT, loss = pl.pallas_call(
        _tc_body,
        grid=(NUM_BLOCKS,),
        in_specs=[
            pl.BlockSpec((D, B), lambda i: (0, 0)),
            pl.BlockSpec((D, TILE_R), lambda i: (0, i)),
            pl.BlockSpec((D, B), lambda i: (0, 0)),
        ],
        out_specs=[
            pl.BlockSpec((TILE_R, B), lambda i: (i, 0)),
            pl.BlockSpec((1, 1), lambda i: (0, 0)),
        ],
        out_shape=[
            jax.ShapeDtypeStruct((C, B), jnp.float32),
            jax.ShapeDtypeStruct((1, 1), jnp.float32),
        ],
        scratch_shapes=[
            pltpu.VMEM((1, B), jnp.float32),
            pltpu.VMEM((1, B), jnp.float32),
        ],
        compiler_params=pltpu.CompilerParams(
            dimension_semantics=("arbitrary",)
        ),
    )(inputs.T, lut.T, tgt_rows.T)

    return (loss[0, 0], logitsT.T)


# speedup vs baseline: 2.5310x; 1.0055x over previous
import functools

import jax
import jax.numpy as jnp
from jax import lax
from jax.experimental import pallas as pl
from jax.experimental.pallas import tpu as pltpu
from jax.experimental.pallas import tpu_sc as plsc

B = 1024
D = 32
C = 100000
SCALAR_GAIN = 30.0
TILE_R = 2048
NUM_BLOCKS = (C + TILE_R - 1) // TILE_R


@functools.cache
def _make_sc_gather():
    info = plsc.get_sparse_core_info()
    nw = info.num_cores * info.num_subcores
    bpw = B // nw
    mesh = plsc.VectorSubcoreMesh(core_axis_name="c", subcore_axis_name="s")

    @functools.partial(
        pl.kernel,
        mesh=mesh,
        out_type=jax.ShapeDtypeStruct((B, D), jnp.float32),
        scratch_types=[
            pltpu.VMEM((bpw,), jnp.int32),
            pltpu.VMEM((bpw, D), jnp.float32),
            pltpu.SemaphoreType.DMA,
        ],
        compiler_params=pltpu.CompilerParams(use_tc_tiling_on_sc=False),
    )
    def gather_rows(lut_hbm, tgt_hbm, out_hbm, idx_v, rows_v, sem):
        wid = lax.axis_index("s") * info.num_cores + lax.axis_index("c")
        base = wid * bpw
        pltpu.sync_copy(tgt_hbm.at[pl.ds(base, bpw)], idx_v)
        pltpu.async_copy(lut_hbm.at[idx_v], rows_v, sem).wait()
        pltpu.sync_copy(rows_v, out_hbm.at[pl.ds(base, bpw)])

    return gather_rows


def _tc_body(xT_ref, lutT_ref, out_ref, lse_ref, m_ref, s_ref):
    i = pl.program_id(0)
    xT = xT_ref[...]
    lt = lutT_ref[...]
    logitsT = lax.dot_general(
        lt, xT, (((0,), (0,)), ((), ())), preferred_element_type=jnp.float32
    ) * SCALAR_GAIN
    out_ref[...] = logitsT

    row = i * TILE_R + lax.broadcasted_iota(jnp.int32, (TILE_R, 1), 0)
    masked = jnp.where(row < C, logitsT, -jnp.inf)
    tile_max = jnp.max(masked, axis=0, keepdims=True)

    @pl.when(i == 0)
    def _init():
        m_ref[...] = jnp.full((1, B), -jnp.inf, jnp.float32)
        s_ref[...] = jnp.zeros((1, B), jnp.float32)

    m_old = m_ref[...]
    m_new = jnp.maximum(m_old, tile_max)
    s_ref[...] = s_ref[...] * jnp.exp(m_old - m_new) + jnp.sum(
        jnp.exp(masked - m_new), axis=0, keepdims=True
    )
    m_ref[...] = m_new

    @pl.when(i == NUM_BLOCKS - 1)
    def _finish():
        lse_ref[...] = m_ref[...] + jnp.log(s_ref[...])


def _combine_body(lse_ref, xT_ref, tgt_ref, loss_ref):
    tgt_logit = jnp.sum(xT_ref[...] * tgt_ref[...], axis=0, keepdims=True) * (
        SCALAR_GAIN
    )
    loss_ref[...] = jnp.mean(lse_ref[...] - tgt_logit, keepdims=True)


def kernel(inputs, targets, lut):
    tgt_rows = _make_sc_gather()(lut, targets)

    logitsT, lse = pl.pallas_call(
        _tc_body,
        grid=(NUM_BLOCKS,),
        in_specs=[
            pl.BlockSpec((D, B), lambda i: (0, 0)),
            pl.BlockSpec((D, TILE_R), lambda i: (0, i)),
        ],
        out_specs=[
            pl.BlockSpec((TILE_R, B), lambda i: (i, 0)),
            pl.BlockSpec((1, B), lambda i: (0, 0)),
        ],
        out_shape=[
            jax.ShapeDtypeStruct((C, B), jnp.float32),
            jax.ShapeDtypeStruct((1, B), jnp.float32),
        ],
        scratch_shapes=[
            pltpu.VMEM((1, B), jnp.float32),
            pltpu.VMEM((1, B), jnp.float32),
        ],
        compiler_params=pltpu.CompilerParams(
            dimension_semantics=("arbitrary",)
        ),
    )(inputs.T, lut.T)

    loss = pl.pallas_call(
        _combine_body,
        in_specs=[
            pl.BlockSpec((1, B), lambda: (0, 0)),
            pl.BlockSpec((D, B), lambda: (0, 0)),
            pl.BlockSpec((D, B), lambda: (0, 0)),
        ],
        out_specs=pl.BlockSpec((1, 1), lambda: (0, 0)),
        out_shape=jax.ShapeDtypeStruct((1, 1), jnp.float32),
    )(lse, inputs.T, tgt_rows.T)

    return (loss[0, 0], logitsT.T)


# speedup vs baseline: 3.1633x; 1.2498x over previous
import functools

import jax
import jax.numpy as jnp
from jax import lax
from jax.experimental import pallas as pl
from jax.experimental.pallas import tpu as pltpu
from jax.experimental.pallas import tpu_sc as plsc

B = 1024
D = 32
C = 100000
SCALAR_GAIN = 30.0
TILE_R = 2048
NUM_BLOCKS = (C + TILE_R - 1) // TILE_R


@functools.cache
def _make_sc_diag_gather():
    info = plsc.get_sparse_core_info()
    nw = info.num_cores * info.num_subcores
    bpw = B // nw
    mesh = plsc.VectorSubcoreMesh(core_axis_name="c", subcore_axis_name="s")

    @functools.partial(
        pl.kernel,
        mesh=mesh,
        out_type=jax.ShapeDtypeStruct((B, B), jnp.float32),
        scratch_types=[
            pltpu.VMEM((bpw,), jnp.int32),
            pltpu.VMEM((bpw, B), jnp.float32),
            pltpu.SemaphoreType.DMA,
        ],
    )
    def gather_rows(lt_hbm, tgt_hbm, out_hbm, idx_v, rows_v, sem):
        wid = lax.axis_index("s") * info.num_cores + lax.axis_index("c")
        base = wid * bpw
        pltpu.sync_copy(tgt_hbm.at[pl.ds(base, bpw)], idx_v)
        pltpu.async_copy(lt_hbm.at[idx_v], rows_v, sem).wait()
        pltpu.sync_copy(rows_v, out_hbm.at[pl.ds(base, bpw)])

    return gather_rows


def _tc_body(xT_ref, lutT_ref, out_ref, lse_ref, m_ref, s_ref):
    i = pl.program_id(0)
    xT = xT_ref[...]
    lt = lutT_ref[...]
    logitsT = lax.dot_general(
        lt, xT, (((0,), (0,)), ((), ())), preferred_element_type=jnp.float32
    ) * SCALAR_GAIN
    out_ref[...] = logitsT

    row = i * TILE_R + lax.broadcasted_iota(jnp.int32, (TILE_R, 1), 0)
    masked = jnp.where(row < C, logitsT, -jnp.inf)
    tile_max = jnp.max(masked, axis=0, keepdims=True)

    @pl.when(i == 0)
    def _init():
        m_ref[...] = jnp.full((1, B), -jnp.inf, jnp.float32)
        s_ref[...] = jnp.zeros((1, B), jnp.float32)

    m_old = m_ref[...]
    m_new = jnp.maximum(m_old, tile_max)
    s_ref[...] = s_ref[...] * jnp.exp(m_old - m_new) + jnp.sum(
        jnp.exp(masked - m_new), axis=0, keepdims=True
    )
    m_ref[...] = m_new

    @pl.when(i == NUM_BLOCKS - 1)
    def _finish():
        lse_ref[...] = m_ref[...] + jnp.log(s_ref[...])


def _combine_body(lse_ref, rows_ref, loss_ref):
    r = lax.broadcasted_iota(jnp.int32, (B, B), 0)
    c = lax.broadcasted_iota(jnp.int32, (B, B), 1)
    tgt_logit = jnp.sum(
        jnp.where(r == c, rows_ref[...], 0.0), axis=0, keepdims=True
    )
    loss_ref[...] = jnp.mean(lse_ref[...] - tgt_logit, keepdims=True)


def kernel(inputs, targets, lut):
    logitsT, lse = pl.pallas_call(
        _tc_body,
        grid=(NUM_BLOCKS,),
        in_specs=[
            pl.BlockSpec((D, B), lambda i: (0, 0)),
            pl.BlockSpec((D, TILE_R), lambda i: (0, i)),
        ],
        out_specs=[
            pl.BlockSpec((TILE_R, B), lambda i: (i, 0)),
            pl.BlockSpec((1, B), lambda i: (0, 0)),
        ],
        out_shape=[
            jax.ShapeDtypeStruct((C, B), jnp.float32),
            jax.ShapeDtypeStruct((1, B), jnp.float32),
        ],
        scratch_shapes=[
            pltpu.VMEM((1, B), jnp.float32),
            pltpu.VMEM((1, B), jnp.float32),
        ],
        compiler_params=pltpu.CompilerParams(
            dimension_semantics=("arbitrary",)
        ),
    )(inputs.T, lut.T)

    tgt_mat = _make_sc_diag_gather()(logitsT, targets)

    loss = pl.pallas_call(
        _combine_body,
        in_specs=[
            pl.BlockSpec((1, B), lambda: (0, 0)),
            pl.BlockSpec((B, B), lambda: (0, 0)),
        ],
        out_specs=pl.BlockSpec((1, 1), lambda: (0, 0)),
        out_shape=jax.ShapeDtypeStruct((1, 1), jnp.float32),
    )(lse, tgt_mat)

    return (loss[0, 0], logitsT.T)
